# Initial kernel scaffold; baseline (speedup 1.0000x reference)
import jax
import jax.numpy as jnp
from jax.experimental import pallas as pl


def kernel(x, weight):
    raise NotImplementedError("write your pallas kernel here")


# speedup vs baseline: 1.9781x; 1.9781x over previous
import functools

import jax
import jax.numpy as jnp
from jax import lax
from jax.experimental import pallas as pl
from jax.experimental.pallas import tpu as pltpu
from jax.experimental.pallas import tpu_sc as plsc

NUM_BINS = 100000
DIM = 32
B = 16384
N = 100

NC = 2
NS = 16
NW = NC * NS

BTOT = B * N
C = 128
K = 16
ROWS_PER_W = BTOT // NW
CHUNKS_PER_W = ROWS_PER_W // C
S = CHUNKS_PER_W // K

_mesh = plsc.VectorSubcoreMesh(core_axis_name="c", subcore_axis_name="s")


@functools.partial(
    pl.kernel,
    out_type=jax.ShapeDtypeStruct((BTOT, DIM), jnp.float32),
    mesh=_mesh,
    scratch_types=[
        pltpu.VMEM((K, C), jnp.int32),
        pltpu.VMEM((K * C, DIM), jnp.float32),
        pltpu.SemaphoreType.DMA,
    ],
    compiler_params=pltpu.CompilerParams(use_tc_tiling_on_sc=False),
)
def _emb_lookup(table_hbm, idx_hbm, out_hbm, idx_v, rows_v, sem):
    wid = lax.axis_index("s") * NC + lax.axis_index("c")
    chunk_base = wid * CHUNKS_PER_W

    def body(g, carry):
        rbase = chunk_base + g * K
        pltpu.sync_copy(idx_hbm.at[pl.ds(rbase, K)], idx_v)
        copies = [
            pltpu.async_copy(
                table_hbm.at[idx_v.at[j]],
                rows_v.at[pl.ds(j * C, C)],
                sem,
            )
            for j in range(K)
        ]
        for cp in copies:
            cp.wait()
        pltpu.sync_copy(rows_v, out_hbm.at[pl.ds(rbase * C, K * C)])
        return carry

    lax.fori_loop(0, S, body, 0)


def kernel(x, weight):
    idx = x.reshape(BTOT // C, C).astype(jnp.int32)
    out = _emb_lookup(weight, idx)
    return out.reshape(B, N, DIM)


# speedup vs baseline: 1.9858x; 1.0039x over previous
import functools

import jax
import jax.numpy as jnp
from jax import lax
from jax.experimental import pallas as pl
from jax.experimental.pallas import tpu as pltpu
from jax.experimental.pallas import tpu_sc as plsc

NUM_BINS = 100000
DIM = 32
B = 16384
N = 100

NC = 2
NS = 16
NW = NC * NS

BTOT = B * N
C = 128
K = 10
NBUF = 2
ROWS_PER_W = BTOT // NW
CHUNKS_PER_W = ROWS_PER_W // C
S = CHUNKS_PER_W // K

_mesh = plsc.VectorSubcoreMesh(core_axis_name="c", subcore_axis_name="s")


@functools.partial(
    pl.kernel,
    out_type=jax.ShapeDtypeStruct((BTOT, DIM), jnp.float32),
    mesh=_mesh,
    scratch_types=[
        pltpu.VMEM((NBUF, K, C), jnp.int32),
        pltpu.VMEM((NBUF, K * C, DIM), jnp.float32),
        [pltpu.SemaphoreType.DMA] * NBUF,
        [pltpu.SemaphoreType.DMA] * NBUF,
    ],
    compiler_params=pltpu.CompilerParams(use_tc_tiling_on_sc=False),
)
def _emb_lookup(table_hbm, idx_hbm, out_hbm, idx_v, rows_v, gsem, osem):
    wid = lax.axis_index("s") * NC + lax.axis_index("c")
    chunk_base = wid * CHUNKS_PER_W

    def fire_gathers(b, g):
        return [
            pltpu.async_copy(
                table_hbm.at[idx_v.at[b].at[j]],
                rows_v.at[b].at[pl.ds(j * C, C)],
                gsem[b],
            )
            for j in range(K)
        ]

    for b in range(NBUF):
        pltpu.sync_copy(idx_hbm.at[pl.ds(chunk_base + b * K, K)], idx_v.at[b])
        fire_gathers(b, b)

    def body(go, carry):
        for b in range(NBUF):
            g = go * NBUF + b
            rbase = chunk_base + g * K
            for j in range(K):
                pltpu.make_async_copy(
                    table_hbm.at[idx_v.at[b].at[j]],
                    rows_v.at[b].at[pl.ds(j * C, C)],
                    gsem[b],
                ).wait()
            ocp = pltpu.async_copy(
                rows_v.at[b], out_hbm.at[pl.ds(rbase * C, K * C)], osem[b]
            )
            gn = g + NBUF

            @pl.when(gn < S)
            def _():
                pltpu.sync_copy(
                    idx_hbm.at[pl.ds(chunk_base + gn * K, K)], idx_v.at[b]
                )

            ocp.wait()

            @pl.when(gn < S)
            def _():
                fire_gathers(b, gn)

        return carry

    lax.fori_loop(0, S // NBUF, body, 0)


def kernel(x, weight):
    idx = x.reshape(BTOT // C, C).astype(jnp.int32)
    out = _emb_lookup(weight, idx)
    return out.reshape(B, N, DIM)


# speedup vs baseline: 10.2423x; 5.1578x over previous
import functools

import jax
import jax.numpy as jnp
from jax import lax
from jax.experimental import pallas as pl
from jax.experimental.pallas import tpu as pltpu
from jax.experimental.pallas import tpu_sc as plsc

NUM_BINS = 100000
DIM = 32
B = 16384
N = 100

NC = 2
NS = 16
NW = NC * NS

KB = 16
NBUF = 2
B_PER_W = B // NW
S = B_PER_W // KB

_mesh = plsc.VectorSubcoreMesh(core_axis_name="c", subcore_axis_name="s")


@functools.partial(
    pl.kernel,
    out_type=jax.ShapeDtypeStruct((B, N, DIM), jnp.float32),
    mesh=_mesh,
    scratch_types=[
        pltpu.VMEM((NBUF, KB, N), jnp.int32),
        pltpu.VMEM((NBUF, KB, N, DIM), jnp.float32),
        [pltpu.SemaphoreType.DMA] * NBUF,
        [pltpu.SemaphoreType.DMA] * NBUF,
    ],
    compiler_params=pltpu.CompilerParams(use_tc_tiling_on_sc=False),
)
def _emb_lookup(table_hbm, idx_hbm, out_hbm, idx_v, rows_v, gsem, osem):
    wid = lax.axis_index("s") * NC + lax.axis_index("c")
    batch_base = wid * B_PER_W

    def fire_gathers(b):
        return [
            pltpu.async_copy(
                table_hbm.at[idx_v.at[b].at[i]],
                rows_v.at[b].at[i],
                gsem[b],
            )
            for i in range(KB)
        ]

    for b in range(NBUF):
        pltpu.sync_copy(idx_hbm.at[pl.ds(batch_base + b * KB, KB)], idx_v.at[b])
        fire_gathers(b)

    def body(go, carry):
        for b in range(NBUF):
            g = go * NBUF + b
            b0 = batch_base + g * KB
            for i in range(KB):
                pltpu.make_async_copy(
                    table_hbm.at[idx_v.at[b].at[i]],
                    rows_v.at[b].at[i],
                    gsem[b],
                ).wait()
            ocp = pltpu.async_copy(
                rows_v.at[b], out_hbm.at[pl.ds(b0, KB)], osem[b]
            )
            gn = g + NBUF

            @pl.when(gn < S)
            def _():
                pltpu.sync_copy(
                    idx_hbm.at[pl.ds(batch_base + gn * KB, KB)], idx_v.at[b]
                )

            ocp.wait()

            @pl.when(gn < S)
            def _():
                fire_gathers(b)

        return carry

    lax.fori_loop(0, S // NBUF, body, 0)


def kernel(x, weight):
    return _emb_lookup(weight, x.astype(jnp.int32))
